# baseline (speedup 1.0000x reference)
import jax
import jax.numpy as jnp
from jax import lax
from jax.experimental import pallas as pl
from jax.experimental.pallas import tpu as pltpu
from jax.experimental.pallas import tpu_sc as plsc

N = 10000
E = 320000
D = 128
LABELS = 64
BM = 2000

NC = 2
NS = 16
NW = NC * NS
EPW = E // NW

C = 125
NCH = EPW // C
RPT = N // NS
RCH = RPT // C
DW = 128

CB = 64
NCHB = -(-EPW // CB)
EPAD = NW * NCHB * CB
DI = D // 2
NPAD = 10048
ZPT = NPAD // NS
TRASH = N

WRT = 640
WLAST_BASE = WRT * (NS - 1)
WLAST = N - WLAST_BASE


def _make_sc_deg():
    def body(dst_hbm, deg_out, dst_v, ones_v, deg_sh, sem):
        cid = lax.axis_index("c")
        sid = lax.axis_index("s")
        wid = cid * NS + sid
        pltpu.sync_copy(dst_hbm.at[wid], dst_v)

        zv = jnp.zeros((16,), jnp.float32)

        def zrow(r, _):
            for col in range(DW // 16):
                ones_v[r, pl.ds(col * 16, 16)] = zv
            return 0

        lax.fori_loop(0, C, zrow, 0)
        base = sid * RPT
        for i in range(RCH):
            pltpu.sync_copy(ones_v, deg_sh.at[pl.ds(base + i * C, C)])

        ov = jnp.ones((16,), jnp.float32)

        def orow(r, _):
            for col in range(DW // 16):
                ones_v[r, pl.ds(col * 16, 16)] = ov
            return 0

        lax.fori_loop(0, C, orow, 0)
        plsc.subcore_barrier()

        def chunk(j, _):
            pltpu.sync_copy(ones_v, deg_sh.at[dst_v.at[j]], add=True)
            return 0

        lax.fori_loop(0, NCH, chunk, 0)
        plsc.subcore_barrier()

        wbase = sid * WRT

        @pl.when(sid < NS - 1)
        def _():
            pltpu.sync_copy(deg_sh.at[pl.ds(wbase, WRT)],
                            deg_out.at[cid, pl.ds(wbase, WRT)])

        @pl.when(sid == NS - 1)
        def _():
            pltpu.sync_copy(deg_sh.at[pl.ds(WLAST_BASE, WLAST)],
                            deg_out.at[cid, pl.ds(WLAST_BASE, WLAST)])

    mesh = plsc.VectorSubcoreMesh(core_axis_name="c", subcore_axis_name="s",
                                  num_cores=NC, num_subcores=NS)
    return pl.kernel(
        body,
        out_type=jax.ShapeDtypeStruct((NC, N, DW), jnp.float32),
        mesh=mesh,
        scratch_types=[
            pltpu.VMEM((NCH, C), jnp.int32),
            pltpu.VMEM((C, DW), jnp.float32),
            pltpu.VMEM_SHARED((N, DW), jnp.float32),
            pltpu.SemaphoreType.DMA,
        ],
        name="sc_deg")


def _make_sc_agg():
    out_type = [jax.ShapeDtypeStruct((NC, N, D), jnp.float32)]
    scratch = [
        pltpu.VMEM((NCHB, CB), jnp.int32),
        pltpu.VMEM((1, CB), jnp.int32),
        pltpu.VMEM((1, CB), jnp.int32),
        pltpu.VMEM((1, CB), jnp.int32),
        pltpu.VMEM((1, CB), jnp.int32),
        pltpu.VMEM((CB, DI), jnp.int32),
        pltpu.VMEM((CB, DI), jnp.int32),
        pltpu.VMEM((CB, D), jnp.float32),
        pltpu.VMEM((CB, D), jnp.float32),
        pltpu.VMEM_SHARED((NPAD, D), jnp.float32),
        pltpu.SemaphoreType.DMA,
        pltpu.SemaphoreType.DMA,
        pltpu.SemaphoreType.DMA,
        pltpu.SemaphoreType.DMA,
    ]

    def body(y_hbm, packed_hbm, *rest):
        (agg_out, packed_v, srcc0, srcc1, dstc0, dstc1,
         gi0, gi1, gf0, gf1, agg_sh, semg0, semg1, sems0, sems1) = rest
        cid = lax.axis_index("c")
        sid = lax.axis_index("s")
        wid = cid * NS + sid

        pltpu.sync_copy(packed_hbm.at[wid], packed_v)

        zv = jnp.zeros((16,), jnp.float32)

        def zrow(r, _):
            for col in range(D // 16):
                gf0[r, pl.ds(col * 16, 16)] = zv
            return 0

        lax.fori_loop(0, CB, zrow, 0)

        base = sid * ZPT
        for i in range(ZPT // CB):
            pltpu.sync_copy(gf0, agg_sh.at[pl.ds(base + i * CB, CB)])
        rem = ZPT % CB
        if rem:
            pltpu.sync_copy(gf0.at[pl.ds(0, rem)],
                            agg_sh.at[pl.ds(base + ZPT - rem, rem)])

        plsc.subcore_barrier()

        srccs = (srcc0, srcc1)
        dstcs = (dstc0, dstc1)
        gis = (gi0, gi1)
        gfs = (gf0, gf1)
        semgs = (semg0, semg1)
        semss = (sems0, sems1)


        def unpack(c, b):
            for k in range(CB // 16):
                p = packed_v[c, pl.ds(k * 16, 16)]
                srccs[b][0, pl.ds(k * 16, 16)] = p >> 14
                dstcs[b][0, pl.ds(k * 16, 16)] = p & 16383

        def widen(b):
            gi, gf = gis[b], gfs[b]

            def wrow(r, _):
                for g in range(DI // 16):
                    v = gi[r, pl.ds(g * 16, 16)]
                    gf[r, pl.ds(g * 32, 16)] = lax.bitcast_convert_type(
                        v << 16, jnp.float32)
                    gf[r, pl.ds(g * 32 + 16, 16)] = lax.bitcast_convert_type(
                        v & jnp.int32(-65536), jnp.float32)
                return 0

            lax.fori_loop(0, CB, wrow, 0)

        unpack(0, 0)
        pltpu.async_copy(y_hbm.at[srcc0.at[0]], gi0, semg0)

        def step(c, b, nxt):
            pltpu.make_async_copy(y_hbm.at[srccs[b].at[0]],
                                  gis[b], semgs[b]).wait()

            @pl.when(c >= 1)
            def _():
                pltpu.make_async_copy(
                    gfs[nxt], agg_sh.at[dstcs[nxt].at[0]], semss[nxt]).wait()

            @pl.when(c + 1 < NCHB)
            def _():
                unpack(c + 1, nxt)
                pltpu.async_copy(y_hbm.at[srccs[nxt].at[0]],
                                 gis[nxt], semgs[nxt])

            widen(b)
            pltpu.async_copy(gfs[b], agg_sh.at[dstcs[b].at[0]],
                             semss[b], add=True)

        def pair(g, _):
            for b in range(2):
                step(2 * g + b, b, (b + 1) % 2)
            return 0

        lax.fori_loop(0, NCHB // 2, pair, 0)
        if NCHB % 2:
            step(NCHB - 1, 0, 1)
        pltpu.make_async_copy(gfs[(NCHB - 1) % 2],
                              agg_sh.at[dstcs[(NCHB - 1) % 2].at[0]],
                              semss[(NCHB - 1) % 2]).wait()

        plsc.subcore_barrier()

        wbase = sid * WRT

        @pl.when(sid < NS - 1)
        def _():
            pltpu.sync_copy(agg_sh.at[pl.ds(wbase, WRT)],
                            agg_out.at[cid, pl.ds(wbase, WRT)])

        @pl.when(sid == NS - 1)
        def _():
            pltpu.sync_copy(agg_sh.at[pl.ds(WLAST_BASE, WLAST)],
                            agg_out.at[cid, pl.ds(WLAST_BASE, WLAST)])

    mesh = plsc.VectorSubcoreMesh(core_axis_name="c", subcore_axis_name="s",
                                  num_cores=NC, num_subcores=NS)
    return pl.kernel(body, out_type=tuple(out_type), mesh=mesh,
                     scratch_types=scratch, name="sc_seg_sum",
                     compiler_params=pltpu.CompilerParams(
                         use_tc_tiling_on_sc=False))


_sc_deg = _make_sc_deg()
_sc_agg = _make_sc_agg()

_PREC = lax.Precision.HIGHEST


def _pre_body(x_ref, wl_ref, y_ref):
    y_ref[...] = jnp.dot(x_ref[...], wl_ref[...].T,
                         preferred_element_type=jnp.float32, precision=_PREC)


def _elu(z):
    return jnp.where(z > 0, z, jnp.exp(jnp.minimum(z, 0.0)) - 1.0)


def _mid_body(agg_ref, deg_ref, x_ref, wr_ref, bl_ref, wln_ref, h_ref, y2_ref):
    deg = deg_ref[0, :, 0:1] + deg_ref[1, :, 0:1]
    rdeg = 1.0 / jnp.maximum(deg, 1.0)
    mean = (agg_ref[0] + agg_ref[1]) * rdeg
    root = jnp.dot(x_ref[...], wr_ref[...].T,
                   preferred_element_type=jnp.float32, precision=_PREC)
    h = _elu(mean + bl_ref[...] + root)
    h_ref[...] = h
    y2_ref[...] = jnp.dot(h, wln_ref[...].T,
                          preferred_element_type=jnp.float32, precision=_PREC)


def _fin_body(agg_ref, deg_ref, h_ref, wr_ref, bl_ref, wf_ref, bf_ref, o_ref):
    deg = deg_ref[0, :, 0:1] + deg_ref[1, :, 0:1]
    rdeg = 1.0 / jnp.maximum(deg, 1.0)
    mean = (agg_ref[0] + agg_ref[1]) * rdeg
    root = jnp.dot(h_ref[...], wr_ref[...].T,
                   preferred_element_type=jnp.float32, precision=_PREC)
    h2 = _elu(mean + bl_ref[...] + root)
    logits = jnp.dot(h2, wf_ref[...].T,
                     preferred_element_type=jnp.float32, precision=_PREC)
    o_ref[...] = jax.nn.sigmoid(logits + bf_ref[...])


_SIGMA = [32 * (c // 32) + 16 * (c % 2) + (c % 32) // 2 for c in range(D)]


def _pack_bf16(y):
    yb = y.astype(jnp.bfloat16).reshape(N, DI, 2)
    return jax.lax.bitcast_convert_type(yb, jnp.int32)


def kernel(x, edge_index, Wl1, bl1, Wr1, Wl2, bl2, Wr2, Wf, bf):
    src_i = edge_index[0].astype(jnp.int32)
    dst_i = edge_index[1].astype(jnp.int32)
    pad = TRASH + (jnp.arange(EPAD - E, dtype=jnp.int32) % (NPAD - N))
    packed = jnp.concatenate([(src_i << 14) | dst_i, pad]).reshape(
        NW, NCHB, CB)
    dst_d = dst_i.reshape(NW, NCH, C)
    bl1r = bl1.reshape(1, D)
    bl2r = bl2.reshape(1, D)
    bfr = bf.reshape(1, LABELS)

    sig = jnp.asarray(_SIGMA, jnp.int32)
    y1 = pl.pallas_call(
        _pre_body,
        out_shape=jax.ShapeDtypeStruct((N, D), jnp.float32),
    )(x, Wl1[sig])

    deg = _sc_deg(dst_d)[:, :, :8]
    (agg1,) = _sc_agg(_pack_bf16(y1), packed)

    h, y2 = pl.pallas_call(
        _mid_body,
        grid=(N // BM,),
        in_specs=[
            pl.BlockSpec((NC, BM, D), lambda i: (0, i, 0)),
            pl.BlockSpec((NC, BM, 8), lambda i: (0, i, 0)),
            pl.BlockSpec((BM, D), lambda i: (i, 0)),
            pl.BlockSpec((D, D), lambda i: (0, 0)),
            pl.BlockSpec((1, D), lambda i: (0, 0)),
            pl.BlockSpec((D, D), lambda i: (0, 0)),
        ],
        out_specs=[pl.BlockSpec((BM, D), lambda i: (i, 0)),
                   pl.BlockSpec((BM, D), lambda i: (i, 0))],
        out_shape=[jax.ShapeDtypeStruct((N, D), jnp.float32),
                   jax.ShapeDtypeStruct((N, D), jnp.float32)],
    )(agg1, deg, x, Wr1, bl1r, Wl2[sig])

    (agg2,) = _sc_agg(_pack_bf16(y2), packed)

    out = pl.pallas_call(
        _fin_body,
        grid=(N // BM,),
        in_specs=[
            pl.BlockSpec((NC, BM, D), lambda i: (0, i, 0)),
            pl.BlockSpec((NC, BM, 8), lambda i: (0, i, 0)),
            pl.BlockSpec((BM, D), lambda i: (i, 0)),
            pl.BlockSpec((D, D), lambda i: (0, 0)),
            pl.BlockSpec((1, D), lambda i: (0, 0)),
            pl.BlockSpec((LABELS, D), lambda i: (0, 0)),
            pl.BlockSpec((1, LABELS), lambda i: (0, 0)),
        ],
        out_specs=pl.BlockSpec((BM, LABELS), lambda i: (i, 0)),
        out_shape=jax.ShapeDtypeStruct((N, LABELS), jnp.float32),
    )(agg2, deg, h, Wr2, bl2r, Wf, bfr)
    return out


# speedup vs baseline: 1.5079x; 1.5079x over previous
import jax
import jax.numpy as jnp
from jax import lax
from jax.experimental import pallas as pl
from jax.experimental.pallas import tpu as pltpu
from jax.experimental.pallas import tpu_sc as plsc

N = 10000
E = 320000
D = 128
LABELS = 64
BM = 2000

NC = 2
NS = 16
NW = NC * NS
EPW = E // NW

C = 125
NCH = EPW // C
RPT = N // NS
RCH = RPT // C
DW = 128

CB = 125
NCHB = EPW // CB

WRT = 640
WLAST_BASE = WRT * (NS - 1)
WLAST = N - WLAST_BASE


def _make_sc_deg():
    def body(dst_hbm, deg_out, dst_v, ones_v, deg_sh, sem):
        cid = lax.axis_index("c")
        sid = lax.axis_index("s")
        wid = cid * NS + sid
        pltpu.sync_copy(dst_hbm.at[wid], dst_v)

        zv = jnp.zeros((16,), jnp.float32)

        def zrow(r, _):
            for col in range(DW // 16):
                ones_v[r, pl.ds(col * 16, 16)] = zv
            return 0

        lax.fori_loop(0, C, zrow, 0)
        base = sid * RPT
        for i in range(RCH):
            pltpu.sync_copy(ones_v, deg_sh.at[pl.ds(base + i * C, C)])

        ov = jnp.ones((16,), jnp.float32)

        def orow(r, _):
            for col in range(DW // 16):
                ones_v[r, pl.ds(col * 16, 16)] = ov
            return 0

        lax.fori_loop(0, C, orow, 0)
        plsc.subcore_barrier()

        def chunk(j, _):
            pltpu.sync_copy(ones_v, deg_sh.at[dst_v.at[j]], add=True)
            return 0

        lax.fori_loop(0, NCH, chunk, 0)
        plsc.subcore_barrier()

        wbase = sid * WRT

        @pl.when(sid < NS - 1)
        def _():
            pltpu.sync_copy(deg_sh.at[pl.ds(wbase, WRT)],
                            deg_out.at[cid, pl.ds(wbase, WRT)])

        @pl.when(sid == NS - 1)
        def _():
            pltpu.sync_copy(deg_sh.at[pl.ds(WLAST_BASE, WLAST)],
                            deg_out.at[cid, pl.ds(WLAST_BASE, WLAST)])

    mesh = plsc.VectorSubcoreMesh(core_axis_name="c", subcore_axis_name="s",
                                  num_cores=NC, num_subcores=NS)
    return pl.kernel(
        body,
        out_type=jax.ShapeDtypeStruct((NC, N, DW), jnp.float32),
        mesh=mesh,
        scratch_types=[
            pltpu.VMEM((NCH, C), jnp.int32),
            pltpu.VMEM((C, DW), jnp.float32),
            pltpu.VMEM_SHARED((N, DW), jnp.float32),
            pltpu.SemaphoreType.DMA,
        ],
        name="sc_deg")


def _make_sc_agg():
    out_type = [jax.ShapeDtypeStruct((NC, N, D), jnp.float32)]
    scratch = [
        pltpu.VMEM((NCHB, CB), jnp.int32),
        pltpu.VMEM((NCHB, CB), jnp.int32),
        pltpu.VMEM((CB, D), jnp.float32),
        pltpu.VMEM_SHARED((N, D), jnp.float32),
        pltpu.SemaphoreType.DMA,
    ]

    def body(y_hbm, src_hbm, dst_hbm, *rest):
        agg_out, src_v, dst_v, stage, agg_sh, sem = rest
        cid = lax.axis_index("c")
        sid = lax.axis_index("s")
        wid = cid * NS + sid

        pltpu.sync_copy(src_hbm.at[wid], src_v)
        pltpu.sync_copy(dst_hbm.at[wid], dst_v)

        zv = jnp.zeros((16,), jnp.float32)

        def zrow(r, _):
            for col in range(D // 16):
                stage[r, pl.ds(col * 16, 16)] = zv
            return 0

        lax.fori_loop(0, CB, zrow, 0)

        base = sid * RPT
        for i in range(RPT // CB):
            pltpu.sync_copy(stage, agg_sh.at[pl.ds(base + i * CB, CB)])

        plsc.subcore_barrier()

        def chunk(j, _):
            pltpu.async_copy(y_hbm.at[src_v.at[j]], stage, sem).wait()
            pltpu.sync_copy(stage, agg_sh.at[dst_v.at[j]], add=True)
            return 0

        lax.fori_loop(0, NCHB, chunk, 0)

        plsc.subcore_barrier()

        wbase = sid * WRT

        @pl.when(sid < NS - 1)
        def _():
            pltpu.sync_copy(agg_sh.at[pl.ds(wbase, WRT)],
                            agg_out.at[cid, pl.ds(wbase, WRT)])

        @pl.when(sid == NS - 1)
        def _():
            pltpu.sync_copy(agg_sh.at[pl.ds(WLAST_BASE, WLAST)],
                            agg_out.at[cid, pl.ds(WLAST_BASE, WLAST)])

    mesh = plsc.VectorSubcoreMesh(core_axis_name="c", subcore_axis_name="s",
                                  num_cores=NC, num_subcores=NS)
    return pl.kernel(body, out_type=tuple(out_type), mesh=mesh,
                     scratch_types=scratch, name="sc_seg_sum")


_sc_deg = _make_sc_deg()
_sc_agg = _make_sc_agg()

_PREC = lax.Precision.HIGHEST


def _pre_body(x_ref, wl_ref, y_ref):
    y_ref[...] = jnp.dot(x_ref[...], wl_ref[...].T,
                         preferred_element_type=jnp.float32, precision=_PREC)


def _elu(z):
    return jnp.where(z > 0, z, jnp.exp(jnp.minimum(z, 0.0)) - 1.0)


def _mid_body(agg_ref, deg_ref, x_ref, wr_ref, bl_ref, wln_ref, h_ref, y2_ref):
    deg = deg_ref[0, :, 0:1] + deg_ref[1, :, 0:1]
    rdeg = 1.0 / jnp.maximum(deg, 1.0)
    mean = (agg_ref[0] + agg_ref[1]) * rdeg
    root = jnp.dot(x_ref[...], wr_ref[...].T,
                   preferred_element_type=jnp.float32, precision=_PREC)
    h = _elu(mean + bl_ref[...] + root)
    h_ref[...] = h
    y2_ref[...] = jnp.dot(h, wln_ref[...].T,
                          preferred_element_type=jnp.float32, precision=_PREC)


def _fin_body(agg_ref, deg_ref, h_ref, wr_ref, bl_ref, wf_ref, bf_ref, o_ref):
    deg = deg_ref[0, :, 0:1] + deg_ref[1, :, 0:1]
    rdeg = 1.0 / jnp.maximum(deg, 1.0)
    mean = (agg_ref[0] + agg_ref[1]) * rdeg
    root = jnp.dot(h_ref[...], wr_ref[...].T,
                   preferred_element_type=jnp.float32, precision=_PREC)
    h2 = _elu(mean + bl_ref[...] + root)
    logits = jnp.dot(h2, wf_ref[...].T,
                     preferred_element_type=jnp.float32, precision=_PREC)
    o_ref[...] = jax.nn.sigmoid(logits + bf_ref[...])


def kernel(x, edge_index, Wl1, bl1, Wr1, Wl2, bl2, Wr2, Wf, bf):
    src_i = edge_index[0].astype(jnp.int32)
    dst_i = edge_index[1].astype(jnp.int32)
    src_a = src_i.reshape(NW, NCHB, CB)
    dst_a = dst_i.reshape(NW, NCHB, CB)
    dst_d = dst_i.reshape(NW, NCH, C)
    bl1r = bl1.reshape(1, D)
    bl2r = bl2.reshape(1, D)
    bfr = bf.reshape(1, LABELS)

    y1 = pl.pallas_call(
        _pre_body,
        out_shape=jax.ShapeDtypeStruct((N, D), jnp.float32),
    )(x, Wl1)

    deg = _sc_deg(dst_d)[:, :, :8]
    (agg1,) = _sc_agg(y1, src_a, dst_a)

    h, y2 = pl.pallas_call(
        _mid_body,
        grid=(N // BM,),
        in_specs=[
            pl.BlockSpec((NC, BM, D), lambda i: (0, i, 0)),
            pl.BlockSpec((NC, BM, 8), lambda i: (0, i, 0)),
            pl.BlockSpec((BM, D), lambda i: (i, 0)),
            pl.BlockSpec((D, D), lambda i: (0, 0)),
            pl.BlockSpec((1, D), lambda i: (0, 0)),
            pl.BlockSpec((D, D), lambda i: (0, 0)),
        ],
        out_specs=[pl.BlockSpec((BM, D), lambda i: (i, 0)),
                   pl.BlockSpec((BM, D), lambda i: (i, 0))],
        out_shape=[jax.ShapeDtypeStruct((N, D), jnp.float32),
                   jax.ShapeDtypeStruct((N, D), jnp.float32)],
    )(agg1, deg, x, Wr1, bl1r, Wl2)

    (agg2,) = _sc_agg(y2, src_a, dst_a)

    out = pl.pallas_call(
        _fin_body,
        grid=(N // BM,),
        in_specs=[
            pl.BlockSpec((NC, BM, D), lambda i: (0, i, 0)),
            pl.BlockSpec((NC, BM, 8), lambda i: (0, i, 0)),
            pl.BlockSpec((BM, D), lambda i: (i, 0)),
            pl.BlockSpec((D, D), lambda i: (0, 0)),
            pl.BlockSpec((1, D), lambda i: (0, 0)),
            pl.BlockSpec((LABELS, D), lambda i: (0, 0)),
            pl.BlockSpec((1, LABELS), lambda i: (0, 0)),
        ],
        out_specs=pl.BlockSpec((BM, LABELS), lambda i: (i, 0)),
        out_shape=jax.ShapeDtypeStruct((N, LABELS), jnp.float32),
    )(agg2, deg, h, Wr2, bl2r, Wf, bfr)
    return out


# speedup vs baseline: 1.6396x; 1.0873x over previous
import jax
import jax.numpy as jnp
from jax import lax
from jax.experimental import pallas as pl
from jax.experimental.pallas import tpu as pltpu
from jax.experimental.pallas import tpu_sc as plsc

N = 10000
E = 320000
D = 128
LABELS = 64
BM = 2000

NC = 2
NS = 16
NW = NC * NS
EPW = E // NW

C = 125
NCH = EPW // C
RPT = N // NS
RCH = RPT // C
DW = 128

CB = 80
NCHB = EPW // CB

WRT = 640
WLAST_BASE = WRT * (NS - 1)
WLAST = N - WLAST_BASE


def _make_sc_deg():
    def body(dst_hbm, deg_out, dst_v, ones_v, deg_sh, sem):
        cid = lax.axis_index("c")
        sid = lax.axis_index("s")
        wid = cid * NS + sid
        pltpu.sync_copy(dst_hbm.at[wid], dst_v)

        zv = jnp.zeros((16,), jnp.float32)

        def zrow(r, _):
            for col in range(DW // 16):
                ones_v[r, pl.ds(col * 16, 16)] = zv
            return 0

        lax.fori_loop(0, C, zrow, 0)
        base = sid * RPT
        for i in range(RCH):
            pltpu.sync_copy(ones_v, deg_sh.at[pl.ds(base + i * C, C)])

        ov = jnp.ones((16,), jnp.float32)

        def orow(r, _):
            for col in range(DW // 16):
                ones_v[r, pl.ds(col * 16, 16)] = ov
            return 0

        lax.fori_loop(0, C, orow, 0)
        plsc.subcore_barrier()

        def chunk(j, _):
            pltpu.sync_copy(ones_v, deg_sh.at[dst_v.at[j]], add=True)
            return 0

        lax.fori_loop(0, NCH, chunk, 0)
        plsc.subcore_barrier()

        wbase = sid * WRT

        @pl.when(sid < NS - 1)
        def _():
            pltpu.sync_copy(deg_sh.at[pl.ds(wbase, WRT)],
                            deg_out.at[cid, pl.ds(wbase, WRT)])

        @pl.when(sid == NS - 1)
        def _():
            pltpu.sync_copy(deg_sh.at[pl.ds(WLAST_BASE, WLAST)],
                            deg_out.at[cid, pl.ds(WLAST_BASE, WLAST)])

    mesh = plsc.VectorSubcoreMesh(core_axis_name="c", subcore_axis_name="s",
                                  num_cores=NC, num_subcores=NS)
    return pl.kernel(
        body,
        out_type=jax.ShapeDtypeStruct((NC, N, DW), jnp.float32),
        mesh=mesh,
        scratch_types=[
            pltpu.VMEM((NCH, C), jnp.int32),
            pltpu.VMEM((C, DW), jnp.float32),
            pltpu.VMEM_SHARED((N, DW), jnp.float32),
            pltpu.SemaphoreType.DMA,
        ],
        name="sc_deg")


def _make_sc_agg():
    out_type = [jax.ShapeDtypeStruct((NC, N, D), jnp.float32)]
    scratch = [
        pltpu.VMEM((NCHB, CB), jnp.int32),
        pltpu.VMEM((1, CB), jnp.int32),
        pltpu.VMEM((1, CB), jnp.int32),
        pltpu.VMEM((1, CB), jnp.int32),
        pltpu.VMEM((1, CB), jnp.int32),
        pltpu.VMEM((CB, D), jnp.float32),
        pltpu.VMEM((CB, D), jnp.float32),
        pltpu.VMEM_SHARED((N, D), jnp.float32),
        pltpu.SemaphoreType.DMA,
        pltpu.SemaphoreType.DMA,
    ]

    def body(y_hbm, packed_hbm, *rest):
        (agg_out, packed_v, srcc0, srcc1, dstc0, dstc1,
         stage0, stage1, agg_sh, sems0, sems1) = rest
        cid = lax.axis_index("c")
        sid = lax.axis_index("s")
        wid = cid * NS + sid

        pltpu.sync_copy(packed_hbm.at[wid], packed_v)

        zv = jnp.zeros((16,), jnp.float32)

        def zrow(r, _):
            for col in range(D // 16):
                stage0[r, pl.ds(col * 16, 16)] = zv
            return 0

        lax.fori_loop(0, CB, zrow, 0)

        base = sid * RPT
        for i in range(RPT // CB):
            pltpu.sync_copy(stage0, agg_sh.at[pl.ds(base + i * CB, CB)])
        rem = RPT % CB
        if rem:
            pltpu.sync_copy(stage0.at[pl.ds(0, rem)],
                            agg_sh.at[pl.ds(base + RPT - rem, rem)])

        plsc.subcore_barrier()

        srccs = (srcc0, srcc1)
        dstcs = (dstc0, dstc1)
        stages = (stage0, stage1)
        semss = (sems0, sems1)

        def unpack(c, b):
            for k in range(CB // 16):
                p = packed_v[c, pl.ds(k * 16, 16)]
                srccs[b][0, pl.ds(k * 16, 16)] = p >> 14
                dstcs[b][0, pl.ds(k * 16, 16)] = p & 16383

        def pair(g, _):
            for b in range(2):
                c = 2 * g + b

                @pl.when(g >= 1)
                def _():
                    pltpu.make_async_copy(
                        stages[b], agg_sh.at[dstcs[b].at[0]],
                        semss[b]).wait()

                unpack(c, b)
                pltpu.sync_copy(y_hbm.at[srccs[b].at[0]], stages[b])
                pltpu.async_copy(stages[b], agg_sh.at[dstcs[b].at[0]],
                                 semss[b], add=True)
            return 0

        lax.fori_loop(0, NCHB // 2, pair, 0)
        for b in range(2):
            pltpu.make_async_copy(stages[b], agg_sh.at[dstcs[b].at[0]],
                                  semss[b]).wait()

        plsc.subcore_barrier()

        wbase = sid * WRT

        @pl.when(sid < NS - 1)
        def _():
            pltpu.sync_copy(agg_sh.at[pl.ds(wbase, WRT)],
                            agg_out.at[cid, pl.ds(wbase, WRT)])

        @pl.when(sid == NS - 1)
        def _():
            pltpu.sync_copy(agg_sh.at[pl.ds(WLAST_BASE, WLAST)],
                            agg_out.at[cid, pl.ds(WLAST_BASE, WLAST)])

    mesh = plsc.VectorSubcoreMesh(core_axis_name="c", subcore_axis_name="s",
                                  num_cores=NC, num_subcores=NS)
    return pl.kernel(body, out_type=tuple(out_type), mesh=mesh,
                     scratch_types=scratch, name="sc_seg_sum")


_sc_deg = _make_sc_deg()
_sc_agg = _make_sc_agg()

_PREC = lax.Precision.HIGHEST


def _pre_body(x_ref, wl_ref, y_ref):
    y_ref[...] = jnp.dot(x_ref[...], wl_ref[...].T,
                         preferred_element_type=jnp.float32, precision=_PREC)


def _elu(z):
    return jnp.where(z > 0, z, jnp.exp(jnp.minimum(z, 0.0)) - 1.0)


def _mid_body(agg_ref, deg_ref, x_ref, wr_ref, bl_ref, wln_ref, h_ref, y2_ref):
    deg = deg_ref[0, :, 0:1] + deg_ref[1, :, 0:1]
    rdeg = 1.0 / jnp.maximum(deg, 1.0)
    mean = (agg_ref[0] + agg_ref[1]) * rdeg
    root = jnp.dot(x_ref[...], wr_ref[...].T,
                   preferred_element_type=jnp.float32, precision=_PREC)
    h = _elu(mean + bl_ref[...] + root)
    h_ref[...] = h
    y2_ref[...] = jnp.dot(h, wln_ref[...].T,
                          preferred_element_type=jnp.float32, precision=_PREC)


def _fin_body(agg_ref, deg_ref, h_ref, wr_ref, bl_ref, wf_ref, bf_ref, o_ref):
    deg = deg_ref[0, :, 0:1] + deg_ref[1, :, 0:1]
    rdeg = 1.0 / jnp.maximum(deg, 1.0)
    mean = (agg_ref[0] + agg_ref[1]) * rdeg
    root = jnp.dot(h_ref[...], wr_ref[...].T,
                   preferred_element_type=jnp.float32, precision=_PREC)
    h2 = _elu(mean + bl_ref[...] + root)
    logits = jnp.dot(h2, wf_ref[...].T,
                     preferred_element_type=jnp.float32, precision=_PREC)
    o_ref[...] = jax.nn.sigmoid(logits + bf_ref[...])


def kernel(x, edge_index, Wl1, bl1, Wr1, Wl2, bl2, Wr2, Wf, bf):
    src_i = edge_index[0].astype(jnp.int32)
    dst_i = edge_index[1].astype(jnp.int32)
    packed = ((src_i << 14) | dst_i).reshape(NW, NCHB, CB)
    dst_d = dst_i.reshape(NW, NCH, C)
    bl1r = bl1.reshape(1, D)
    bl2r = bl2.reshape(1, D)
    bfr = bf.reshape(1, LABELS)

    y1 = pl.pallas_call(
        _pre_body,
        out_shape=jax.ShapeDtypeStruct((N, D), jnp.float32),
    )(x, Wl1)

    deg = _sc_deg(dst_d)[:, :, :8]
    (agg1,) = _sc_agg(y1, packed)

    h, y2 = pl.pallas_call(
        _mid_body,
        grid=(N // BM,),
        in_specs=[
            pl.BlockSpec((NC, BM, D), lambda i: (0, i, 0)),
            pl.BlockSpec((NC, BM, 8), lambda i: (0, i, 0)),
            pl.BlockSpec((BM, D), lambda i: (i, 0)),
            pl.BlockSpec((D, D), lambda i: (0, 0)),
            pl.BlockSpec((1, D), lambda i: (0, 0)),
            pl.BlockSpec((D, D), lambda i: (0, 0)),
        ],
        out_specs=[pl.BlockSpec((BM, D), lambda i: (i, 0)),
                   pl.BlockSpec((BM, D), lambda i: (i, 0))],
        out_shape=[jax.ShapeDtypeStruct((N, D), jnp.float32),
                   jax.ShapeDtypeStruct((N, D), jnp.float32)],
    )(agg1, deg, x, Wr1, bl1r, Wl2)

    (agg2,) = _sc_agg(y2, packed)

    out = pl.pallas_call(
        _fin_body,
        grid=(N // BM,),
        in_specs=[
            pl.BlockSpec((NC, BM, D), lambda i: (0, i, 0)),
            pl.BlockSpec((NC, BM, 8), lambda i: (0, i, 0)),
            pl.BlockSpec((BM, D), lambda i: (i, 0)),
            pl.BlockSpec((D, D), lambda i: (0, 0)),
            pl.BlockSpec((1, D), lambda i: (0, 0)),
            pl.BlockSpec((LABELS, D), lambda i: (0, 0)),
            pl.BlockSpec((1, LABELS), lambda i: (0, 0)),
        ],
        out_specs=pl.BlockSpec((BM, LABELS), lambda i: (i, 0)),
        out_shape=jax.ShapeDtypeStruct((N, LABELS), jnp.float32),
    )(agg2, deg, h, Wr2, bl2r, Wf, bfr)
    return out
